# baseline (speedup 1.0000x reference)
import functools

import jax
import jax.numpy as jnp
import numpy as np
from jax import lax
from jax.experimental import pallas as pl
from jax.experimental.pallas import tpu as pltpu
from jax.experimental.pallas import tpu_sc as plsc

N_NODES = 10000
N_EDGES = 320000
D = 128
K_PATTERNS = 1024
BETA = 1.0
LAMBDA_GRAPH = 0.1
ALPHA = 0.5
LN_EPS = 1e-5

NC = 2
NS = 16
NW = NC * NS
N_PAD = 10240
CHUNK = 128
NCHUNK = 80
E_PAD = NW * NCHUNK * CHUNK
ROWS_PER_TILE = N_PAD // NS
DEG_W = 16


def _worker_id():
    return lax.axis_index("s") * NC + lax.axis_index("c")


def _deg_body(ei_hbm, deg_out, idx_v, deg_v):
    wid = _worker_id()
    n_per_w = N_EDGES // NW
    pltpu.sync_copy(ei_hbm.at[pl.ds(N_EDGES + wid * n_per_w, n_per_w)], idx_v)

    def zrow(i, carry):
        for k in range(CHUNK // 16):
            deg_v[i, pl.ds(k * 16, 16)] = jnp.zeros((16,), jnp.float32)
        return carry

    lax.fori_loop(0, N_PAD // CHUNK, zrow, 0)

    def hvec(j, carry):
        vec = idx_v[pl.ds(j * 16, 16)]
        cnt, last = plsc.scan_count(vec)
        hi = lax.shift_right_logical(vec, 7)
        lo = lax.bitwise_and(vec, 127)
        plsc.addupdate_scatter(deg_v, [hi, lo], cnt.astype(jnp.float32),
                               mask=last)
        return carry

    lax.fori_loop(0, n_per_w // 16, hvec, 0)
    pltpu.sync_copy(deg_v, deg_out.at[wid])


@functools.cache
def _deg_sc():
    return pl.kernel(
        _deg_body,
        mesh=plsc.VectorSubcoreMesh(core_axis_name="c", subcore_axis_name="s"),
        compiler_params=pltpu.CompilerParams(needs_layout_passes=False),
        out_type=jax.ShapeDtypeStruct((NW, N_PAD // CHUNK, CHUNK), jnp.float32),
        scratch_types=[
            pltpu.VMEM((N_EDGES // NW,), jnp.int32),
            pltpu.VMEM((N_PAD // CHUNK, CHUNK), jnp.float32),
        ],
    )


ACH = 64
NBUF = 4
ANCH_H = E_PAD // NW // ACH // 4


def _agg_body(normx_hbm, row_hbm, col_hbm, zeros_hbm, agg_out, idxr_v, idxc_v,
              rows0, rows1, rows2, rows3, acc_sh,
              sg0, sg1, sg2, sg3, ss0, ss1, ss2, ss3):
    cid = lax.axis_index("c")
    sid = lax.axis_index("s")
    wid = _worker_id()
    rows = (rows0, rows1, rows2, rows3)
    sg = (sg0, sg1, sg2, sg3)
    ss = (ss0, ss1, ss2, ss3)
    pltpu.sync_copy(zeros_hbm, rows0)
    for c in range(ROWS_PER_TILE // ACH):
        pltpu.sync_copy(
            rows0, acc_sh.at[pl.ds(sid * ROWS_PER_TILE + c * ACH, ACH)])
    plsc.subcore_barrier()

    for h in range(4):
        pltpu.sync_copy(row_hbm.at[wid * 4 + h], idxr_v)
        pltpu.sync_copy(col_hbm.at[wid * 4 + h], idxc_v)
        pltpu.async_copy(normx_hbm.at[idxc_v.at[0]], rows0, sg0)
        pltpu.async_copy(normx_hbm.at[idxc_v.at[1]], rows1, sg1)

        def outer(i, carry):
            for b in range(NBUF):
                j = i * NBUF + b
                pltpu.make_async_copy(
                    normx_hbm.at[idxc_v.at[j]], rows[b], sg[b]).wait()
                pltpu.async_copy(
                    rows[b], acc_sh.at[idxr_v.at[j]], ss[b], add=True)
                prv = j - 2
                bp = (b + 2) % NBUF

                @pl.when(prv >= 0)
                def _():
                    pltpu.make_async_copy(
                        rows[bp], acc_sh.at[idxr_v.at[prv]], ss[bp]).wait()
                nxt = j + 2

                @pl.when(nxt < ANCH_H)
                def _():
                    pltpu.async_copy(
                        normx_hbm.at[idxc_v.at[nxt]], rows[bp], sg[bp])
            return carry

        lax.fori_loop(0, ANCH_H // NBUF, outer, 0)
        for j in (ANCH_H - 2, ANCH_H - 1):
            b = j % NBUF
            pltpu.make_async_copy(
                rows[b], acc_sh.at[idxr_v.at[j]], ss[b]).wait()
    plsc.subcore_barrier()
    base = cid * N_PAD + sid * ROWS_PER_TILE
    nd = ROWS_PER_TILE // ACH
    for c in range(nd):
        b = c % NBUF
        if c >= NBUF:
            pltpu.make_async_copy(
                rows[b], agg_out.at[pl.ds(base + (c - NBUF) * ACH, ACH)],
                sg[b]).wait()
        pltpu.sync_copy(
            acc_sh.at[pl.ds(sid * ROWS_PER_TILE + c * ACH, ACH)], rows[b])
        pltpu.async_copy(
            rows[b], agg_out.at[pl.ds(base + c * ACH, ACH)], sg[b])
    for c in range(nd - NBUF, nd):
        pltpu.make_async_copy(
            rows[c % NBUF], agg_out.at[pl.ds(base + c * ACH, ACH)],
            sg[c % NBUF]).wait()


@functools.cache
def _agg_sc():
    return pl.kernel(
        _agg_body,
        mesh=plsc.VectorSubcoreMesh(core_axis_name="c", subcore_axis_name="s"),
        cost_estimate=pl.CostEstimate(
            flops=0, transcendentals=0, bytes_accessed=360_000_000),
        out_type=jax.ShapeDtypeStruct((NC * N_PAD, D), jnp.float32),
        scratch_types=[
            pltpu.VMEM((ANCH_H, ACH), jnp.int32),
            pltpu.VMEM((ANCH_H, ACH), jnp.int32),
            pltpu.VMEM((ACH, D), jnp.float32),
            pltpu.VMEM((ACH, D), jnp.float32),
            pltpu.VMEM((ACH, D), jnp.float32),
            pltpu.VMEM((ACH, D), jnp.float32),
            pltpu.VMEM_SHARED((N_PAD, D), jnp.float32),
            pltpu.SemaphoreType.DMA,
            pltpu.SemaphoreType.DMA,
            pltpu.SemaphoreType.DMA,
            pltpu.SemaphoreType.DMA,
            pltpu.SemaphoreType.DMA,
            pltpu.SemaphoreType.DMA,
            pltpu.SemaphoreType.DMA,
            pltpu.SemaphoreType.DMA,
        ],
    )


def _pre_body(dp_ref, x_ref, normx_ref, dinv_ref):
    deg = jnp.sum(dp_ref[...], axis=0)
    dinv = jnp.where(deg > 0, lax.rsqrt(deg), 0.0)
    normx_ref[...] = dinv[:, None] * x_ref[...]
    dinv_ref[...] = dinv[:, None]


def _pre_tc(deg_parts, x_pad):
    blk = 1280
    grid = N_PAD // blk
    return pl.pallas_call(
        _pre_body,
        grid=(grid,),
        in_specs=[
            pl.BlockSpec((NW, blk), lambda i: (0, i)),
            pl.BlockSpec((blk, D), lambda i: (i, 0)),
        ],
        out_specs=[
            pl.BlockSpec((blk, D), lambda i: (i, 0)),
            pl.BlockSpec((blk, 1), lambda i: (i, 0)),
        ],
        out_shape=[
            jax.ShapeDtypeStruct((N_PAD, D), jnp.float32),
            jax.ShapeDtypeStruct((N_PAD, 1), jnp.float32),
        ],
    )(deg_parts, x_pad)


def _hop_body(x_ref, m_ref, out_ref):
    x = x_ref[...]
    m = m_ref[...]
    sim = BETA * lax.dot_general(x, m, (((1,), (1,)), ((), ())),
                                 preferred_element_type=jnp.float32)
    mx = jnp.max(sim, axis=1, keepdims=True)
    e = jnp.exp(sim - mx)
    attn = e / jnp.sum(e, axis=1, keepdims=True)
    out_ref[...] = lax.dot_general(attn, m, (((1,), (0,)), ((), ())),
                                   preferred_element_type=jnp.float32)


def _hop_tc(x, M):
    blk = 1000
    grid = N_NODES // blk
    return pl.pallas_call(
        _hop_body,
        grid=(grid,),
        compiler_params=pltpu.CompilerParams(skip_device_barrier=True),
        in_specs=[
            pl.BlockSpec((blk, D), lambda i: (i, 0)),
            pl.BlockSpec((K_PATTERNS, D), lambda i: (0, 0)),
        ],
        out_specs=pl.BlockSpec((blk, D), lambda i: (i, 0)),
        out_shape=jax.ShapeDtypeStruct((N_NODES, D), jnp.float32),
    )(x, M)


def _comb_body(x_ref, ret_ref, dinv_ref, op_ref, g_ref, b_ref, out_ref):
    x = x_ref[...]
    retrieved = ret_ref[...]
    dinv = dinv_ref[...]
    out_sum = op_ref[0] + op_ref[1]
    agg = dinv * out_sum
    lap = x - agg
    x_new = (1.0 - ALPHA) * x + ALPHA * (retrieved - 2.0 * LAMBDA_GRAPH * lap)
    mean = jnp.mean(x_new, axis=1, keepdims=True)
    var = jnp.mean((x_new - mean) ** 2, axis=1, keepdims=True)
    xn = (x_new - mean) * lax.rsqrt(var + LN_EPS)
    out_ref[...] = xn * g_ref[...] + b_ref[...]


def _comb_tc(x, retrieved, dinv, out_parts, gamma, beta):
    blk = 1000
    grid = N_NODES // blk
    return pl.pallas_call(
        _comb_body,
        grid=(grid,),
        in_specs=[
            pl.BlockSpec((blk, D), lambda i: (i, 0)),
            pl.BlockSpec((blk, D), lambda i: (i, 0)),
            pl.BlockSpec((blk, 1), lambda i: (i, 0)),
            pl.BlockSpec((NC, blk, D), lambda i: (0, i, 0)),
            pl.BlockSpec((1, D), lambda i: (0, 0)),
            pl.BlockSpec((1, D), lambda i: (0, 0)),
        ],
        out_specs=pl.BlockSpec((blk, D), lambda i: (i, 0)),
        out_shape=jax.ShapeDtypeStruct((N_NODES, D), jnp.float32),
    )(x, retrieved, dinv, out_parts, gamma, beta)


_PAD_IDX = (N_NODES + (np.arange(E_PAD - N_EDGES) % (N_PAD - N_NODES))
            ).astype(np.int32)


def kernel(x, edge_index, M, ln_gamma, ln_beta):
    x = x.astype(jnp.float32)
    edge_index = edge_index.astype(jnp.int32)
    row = edge_index[0]
    col = edge_index[1]
    pad_idx = jnp.asarray(_PAD_IDX)
    row_h = jnp.concatenate([row, pad_idx]).reshape(NW * 4, ANCH_H, ACH)
    col_h = jnp.concatenate([col, pad_idx]).reshape(NW * 4, ANCH_H, ACH)
    x_pad = jnp.pad(x, ((0, N_PAD - N_NODES), (0, 0)))

    zeros_agg = jnp.zeros((ACH, D), jnp.float32)

    deg_parts = _deg_sc()(edge_index.reshape(2 * N_EDGES)).reshape(NW, N_PAD)
    normx, dinv = _pre_tc(deg_parts, x_pad)
    out_parts = _agg_sc()(normx, row_h, col_h, zeros_agg).reshape(NC, N_PAD, D)
    retrieved = _hop_tc(x, M)
    return _comb_tc(x, retrieved, dinv[:N_NODES], out_parts,
                    ln_gamma.reshape(1, D), ln_beta.reshape(1, D))


# speedup vs baseline: 1.1260x; 1.1260x over previous
import functools

import jax
import jax.numpy as jnp
import numpy as np
from jax import lax
from jax.experimental import pallas as pl
from jax.experimental.pallas import tpu as pltpu
from jax.experimental.pallas import tpu_sc as plsc

N_NODES = 10000
N_EDGES = 320000
D = 128
K_PATTERNS = 1024
BETA = 1.0
LAMBDA_GRAPH = 0.1
ALPHA = 0.5
LN_EPS = 1e-5

NC = 2
NS = 16
NW = NC * NS
N_PAD = 10240
CHUNK = 128
NCHUNK = 80
E_PAD = NW * NCHUNK * CHUNK
ROWS_PER_TILE = N_PAD // NS
DEG_W = 16


def _worker_id():
    return lax.axis_index("s") * NC + lax.axis_index("c")


def _deg_body(ei_hbm, deg_out, idx_v, deg_v):
    wid = _worker_id()
    n_per_w = N_EDGES // NW
    pltpu.sync_copy(ei_hbm.at[pl.ds(N_EDGES + wid * n_per_w, n_per_w)], idx_v)

    def zrow(i, carry):
        for k in range(CHUNK // 16):
            deg_v[i, pl.ds(k * 16, 16)] = jnp.zeros((16,), jnp.float32)
        return carry

    lax.fori_loop(0, N_PAD // CHUNK, zrow, 0)

    def hvec(j, carry):
        vec = idx_v[pl.ds(j * 16, 16)]
        cnt, last = plsc.scan_count(vec)
        hi = lax.shift_right_logical(vec, 7)
        lo = lax.bitwise_and(vec, 127)
        plsc.addupdate_scatter(deg_v, [hi, lo], cnt.astype(jnp.float32),
                               mask=last)
        return carry

    lax.fori_loop(0, n_per_w // 16, hvec, 0)
    pltpu.sync_copy(deg_v, deg_out.at[wid])


@functools.cache
def _deg_sc():
    return pl.kernel(
        _deg_body,
        mesh=plsc.VectorSubcoreMesh(core_axis_name="c", subcore_axis_name="s"),
        compiler_params=pltpu.CompilerParams(needs_layout_passes=False),
        out_type=jax.ShapeDtypeStruct((NW, N_PAD // CHUNK, CHUNK), jnp.float32),
        scratch_types=[
            pltpu.VMEM((N_EDGES // NW,), jnp.int32),
            pltpu.VMEM((N_PAD // CHUNK, CHUNK), jnp.float32),
        ],
    )


NCHUNK_H = NCHUNK // 2


def _agg_body(normx_hbm, row_hbm, col_hbm, zeros_hbm, agg_out, idxr_v, idxc_v,
              rows0, rows1, acc_sh, sg0, sg1, ss0, ss1):
    cid = lax.axis_index("c")
    sid = lax.axis_index("s")
    wid = _worker_id()
    rows = (rows0, rows1)
    sg = (sg0, sg1)
    ss = (ss0, ss1)
    pltpu.sync_copy(zeros_hbm, rows0)
    for c in range(ROWS_PER_TILE // CHUNK):
        pltpu.sync_copy(
            rows0, acc_sh.at[pl.ds(sid * ROWS_PER_TILE + c * CHUNK, CHUNK)])
    plsc.subcore_barrier()

    for h in range(2):
        pltpu.sync_copy(row_hbm.at[wid * 2 + h], idxr_v)
        pltpu.sync_copy(col_hbm.at[wid * 2 + h], idxc_v)
        pltpu.async_copy(normx_hbm.at[idxc_v.at[0]], rows0, sg0)
        pltpu.async_copy(normx_hbm.at[idxc_v.at[1]], rows1, sg1)

        def outer(i, carry):
            for b in range(2):
                j = i * 2 + b
                pltpu.make_async_copy(
                    normx_hbm.at[idxc_v.at[j]], rows[b], sg[b]).wait()
                pltpu.async_copy(
                    rows[b], acc_sh.at[idxr_v.at[j]], ss[b], add=True)
                pltpu.make_async_copy(
                    rows[b], acc_sh.at[idxr_v.at[j]], ss[b]).wait()
                nxt = j + 2

                @pl.when(nxt < NCHUNK_H)
                def _():
                    pltpu.async_copy(
                        normx_hbm.at[idxc_v.at[nxt]], rows[b], sg[b])
            return carry

        lax.fori_loop(0, NCHUNK_H // 2, outer, 0)
    plsc.subcore_barrier()
    base = cid * N_PAD + sid * ROWS_PER_TILE
    nd = ROWS_PER_TILE // CHUNK
    for c in range(nd):
        b = c % 2
        if c >= 2:
            pltpu.make_async_copy(
                rows[b], agg_out.at[pl.ds(base + (c - 2) * CHUNK, CHUNK)],
                sg[b]).wait()
        pltpu.sync_copy(
            acc_sh.at[pl.ds(sid * ROWS_PER_TILE + c * CHUNK, CHUNK)], rows[b])
        pltpu.async_copy(
            rows[b], agg_out.at[pl.ds(base + c * CHUNK, CHUNK)], sg[b])
    for c in (nd - 2, nd - 1):
        pltpu.make_async_copy(
            rows[c % 2], agg_out.at[pl.ds(base + c * CHUNK, CHUNK)],
            sg[c % 2]).wait()


@functools.cache
def _agg_sc():
    return pl.kernel(
        _agg_body,
        mesh=plsc.VectorSubcoreMesh(core_axis_name="c", subcore_axis_name="s"),
        cost_estimate=pl.CostEstimate(
            flops=0, transcendentals=0, bytes_accessed=360_000_000),
        out_type=jax.ShapeDtypeStruct((NC * N_PAD, D), jnp.float32),
        scratch_types=[
            pltpu.VMEM((NCHUNK_H, CHUNK), jnp.int32),
            pltpu.VMEM((NCHUNK_H, CHUNK), jnp.int32),
            pltpu.VMEM((CHUNK, D), jnp.float32),
            pltpu.VMEM((CHUNK, D), jnp.float32),
            pltpu.VMEM_SHARED((N_PAD, D), jnp.float32),
            pltpu.SemaphoreType.DMA,
            pltpu.SemaphoreType.DMA,
            pltpu.SemaphoreType.DMA,
            pltpu.SemaphoreType.DMA,
        ],
    )


def _pre_body(dp_ref, x_ref, normx_ref, dinv_ref):
    deg = jnp.sum(dp_ref[...], axis=0)
    dinv = jnp.where(deg > 0, lax.rsqrt(deg), 0.0)
    normx_ref[...] = dinv[:, None] * x_ref[...]
    dinv_ref[...] = dinv[:, None]


def _pre_tc(deg_parts, x_pad):
    blk = 1280
    grid = N_PAD // blk
    return pl.pallas_call(
        _pre_body,
        grid=(grid,),
        in_specs=[
            pl.BlockSpec((NW, blk), lambda i: (0, i)),
            pl.BlockSpec((blk, D), lambda i: (i, 0)),
        ],
        out_specs=[
            pl.BlockSpec((blk, D), lambda i: (i, 0)),
            pl.BlockSpec((blk, 1), lambda i: (i, 0)),
        ],
        out_shape=[
            jax.ShapeDtypeStruct((N_PAD, D), jnp.float32),
            jax.ShapeDtypeStruct((N_PAD, 1), jnp.float32),
        ],
    )(deg_parts, x_pad)


def _hop_body(x_ref, m_ref, out_ref):
    x = x_ref[...]
    m = m_ref[...]
    sim = BETA * lax.dot_general(x, m, (((1,), (1,)), ((), ())),
                                 preferred_element_type=jnp.float32)
    mx = jnp.max(sim, axis=1, keepdims=True)
    e = jnp.exp(sim - mx)
    attn = e / jnp.sum(e, axis=1, keepdims=True)
    out_ref[...] = lax.dot_general(attn, m, (((1,), (0,)), ((), ())),
                                   preferred_element_type=jnp.float32)


def _hop_tc(x, M):
    blk = 1000
    grid = N_NODES // blk
    return pl.pallas_call(
        _hop_body,
        grid=(grid,),
        compiler_params=pltpu.CompilerParams(skip_device_barrier=True),
        in_specs=[
            pl.BlockSpec((blk, D), lambda i: (i, 0)),
            pl.BlockSpec((K_PATTERNS, D), lambda i: (0, 0)),
        ],
        out_specs=pl.BlockSpec((blk, D), lambda i: (i, 0)),
        out_shape=jax.ShapeDtypeStruct((N_NODES, D), jnp.float32),
    )(x, M)


def _comb_body(x_ref, ret_ref, dinv_ref, op_ref, g_ref, b_ref, out_ref):
    x = x_ref[...]
    retrieved = ret_ref[...]
    dinv = dinv_ref[...]
    out_sum = op_ref[0] + op_ref[1]
    agg = dinv * out_sum
    lap = x - agg
    x_new = (1.0 - ALPHA) * x + ALPHA * (retrieved - 2.0 * LAMBDA_GRAPH * lap)
    mean = jnp.mean(x_new, axis=1, keepdims=True)
    var = jnp.mean((x_new - mean) ** 2, axis=1, keepdims=True)
    xn = (x_new - mean) * lax.rsqrt(var + LN_EPS)
    out_ref[...] = xn * g_ref[...] + b_ref[...]


def _comb_tc(x, retrieved, dinv, out_parts, gamma, beta):
    blk = 1000
    grid = N_NODES // blk
    return pl.pallas_call(
        _comb_body,
        grid=(grid,),
        in_specs=[
            pl.BlockSpec((blk, D), lambda i: (i, 0)),
            pl.BlockSpec((blk, D), lambda i: (i, 0)),
            pl.BlockSpec((blk, 1), lambda i: (i, 0)),
            pl.BlockSpec((NC, blk, D), lambda i: (0, i, 0)),
            pl.BlockSpec((1, D), lambda i: (0, 0)),
            pl.BlockSpec((1, D), lambda i: (0, 0)),
        ],
        out_specs=pl.BlockSpec((blk, D), lambda i: (i, 0)),
        out_shape=jax.ShapeDtypeStruct((N_NODES, D), jnp.float32),
    )(x, retrieved, dinv, out_parts, gamma, beta)


_PAD_IDX = (N_NODES + (np.arange(E_PAD - N_EDGES) % (N_PAD - N_NODES))
            ).astype(np.int32)


def kernel(x, edge_index, M, ln_gamma, ln_beta):
    x = x.astype(jnp.float32)
    edge_index = edge_index.astype(jnp.int32)
    row = edge_index[0]
    col = edge_index[1]
    pad_idx = jnp.asarray(_PAD_IDX)
    row_p = jnp.concatenate([row, pad_idx]).reshape(NW, NCHUNK, CHUNK)
    col_p = jnp.concatenate([col, pad_idx]).reshape(NW, NCHUNK, CHUNK)
    row_h = row_p.reshape(NW * 2, NCHUNK_H, CHUNK)
    col_h = col_p.reshape(NW * 2, NCHUNK_H, CHUNK)
    x_pad = jnp.pad(x, ((0, N_PAD - N_NODES), (0, 0)))

    zeros_agg = jnp.zeros((CHUNK, D), jnp.float32)

    deg_parts = _deg_sc()(edge_index.reshape(2 * N_EDGES)).reshape(NW, N_PAD)
    normx, dinv = _pre_tc(deg_parts, x_pad)
    out_parts = _agg_sc()(normx, row_h, col_h, zeros_agg).reshape(NC, N_PAD, D)
    retrieved = _hop_tc(x, M)
    return _comb_tc(x, retrieved, dinv[:N_NODES], out_parts,
                    ln_gamma.reshape(1, D), ln_beta.reshape(1, D))
